# baseline (speedup 1.0000x reference)
import functools
import math

import jax
import jax.numpy as jnp
from jax import lax
from jax.experimental import pallas as pl
from jax.experimental.pallas import tpu as pltpu
from jax.experimental.pallas import tpu_sc as plsc

VOCAB = 1000000
DIM = 64
SCALE = math.sqrt(DIM)
LANES = 16
NUM_CORES = 2
NUM_SUBCORES = 16
NUM_WORKERS = NUM_CORES * NUM_SUBCORES
BBLK = 256
VBLK = 8192


def _pairs_body(tt_ref, p_ref):
    t8 = tt_ref[...].T * SCALE
    r3 = t8.reshape(VBLK // 2, 2, DIM)
    p_ref[...] = jnp.concatenate([r3[:, 0, :], r3[:, 1, :]], axis=1)


def _make_pairs(tt):
    grid = (pl.cdiv(VOCAB, VBLK),)
    return pl.pallas_call(
        _pairs_body,
        grid=grid,
        in_specs=[pl.BlockSpec((DIM, VBLK), lambda j: (0, j))],
        out_specs=pl.BlockSpec((VBLK // 2, 2 * DIM), lambda j: (j, 0)),
        out_shape=jax.ShapeDtypeStruct((VOCAB // 2, 2 * DIM), jnp.float32),
    )(tt)


@functools.lru_cache(maxsize=None)
def _build_lookup(seq: int, batch: int):
    n_bblk = batch // BBLK
    n_units = seq * n_bblk
    assert n_units % NUM_WORKERS == 0
    units_per_w = n_units // NUM_WORKERS
    assert units_per_w % 2 == 0

    mesh = plsc.VectorSubcoreMesh(
        core_axis_name="c", subcore_axis_name="s",
        num_cores=NUM_CORES, num_subcores=NUM_SUBCORES)

    @functools.partial(
        pl.kernel,
        out_type=jax.ShapeDtypeStruct((seq, DIM, batch), jnp.float32),
        mesh=mesh,
        scratch_types=[
            pltpu.VMEM((BBLK,), jnp.int32),
            pltpu.VMEM((BBLK,), jnp.int32),
            pltpu.VMEM((BBLK,), jnp.int32),
            pltpu.VMEM((BBLK,), jnp.int32),
            pltpu.VMEM((BBLK, 2 * DIM), jnp.float32),
            pltpu.VMEM((BBLK, 2 * DIM), jnp.float32),
            pltpu.VMEM((DIM, BBLK), jnp.float32),
            pltpu.VMEM((DIM, BBLK), jnp.float32),
            pltpu.SemaphoreType.DMA,
            pltpu.SemaphoreType.DMA,
            pltpu.SemaphoreType.DMA,
            pltpu.SemaphoreType.DMA,
        ],
        compiler_params=pltpu.CompilerParams(
            use_tc_tiling_on_sc=True, needs_layout_passes=False),
    )
    def lookup_kernel(xt_hbm, p_hbm, out_hbm, idx0, idx1, j0, j1,
                      rows0, rows1, tr0, tr1, gsem0, gsem1, osem0, osem1):
        idx = (idx0, idx1)
        jv = (j0, j1)
        rows = (rows0, rows1)
        tr = (tr0, tr1)
        gsem = (gsem0, gsem1)
        osem = (osem0, osem1)
        wid = lax.axis_index("s") * NUM_CORES + lax.axis_index("c")
        lane = lax.broadcasted_iota(jnp.int32, (LANES,), 0)

        def unit_tb(u):
            unit = wid * units_per_w + u
            return unit // n_bblk, (unit % n_bblk) * BBLK

        def gather_start(u, b):
            t, b0 = unit_tb(u)
            pltpu.sync_copy(xt_hbm.at[t, pl.ds(b0, BBLK)], idx[b])

            @pl.loop(0, BBLK // LANES)
            def _pairify(r):
                sl = pl.ds(r * LANES, LANES)
                jv[b][sl] = lax.shift_right_logical(idx[b][sl], 1)

            pltpu.async_copy(p_hbm.at[jv[b]], rows[b], gsem[b])

        def gather_wait(u, b):
            pltpu.make_async_copy(
                p_hbm.at[jv[b]], rows[b], gsem[b]).wait()

        def out_start(u, b):
            t, b0 = unit_tb(u)
            pltpu.async_copy(
                tr[b], out_hbm.at[t, :, pl.ds(b0, BBLK)], osem[b])

        def out_wait(u, b):
            t, b0 = unit_tb(u)
            pltpu.make_async_copy(
                tr[b], out_hbm.at[t, :, pl.ds(b0, BBLK)], osem[b]).wait()

        def transform(b):
            @plsc.parallel_loop(0, BBLK // LANES, unroll=4)
            def _rows(r):
                sl = pl.ds(r * LANES, LANES)
                rows16 = r * LANES + lane
                half16 = lax.shift_left(
                    lax.bitwise_and(idx[b][sl], jnp.int32(1)), 6)
                for d in range(DIM):
                    tr[b][d, sl] = plsc.load_gather(
                        rows[b], [rows16, half16 + d])

        gather_start(0, 0)

        @pl.loop(0, units_per_w, step=2)
        def _steady(outer):
            for b in range(2):
                u = outer + b
                other = 1 - b

                @pl.when(u > 0)
                def _():
                    out_wait(u - 1, other)

                @pl.when(u + 1 < units_per_w)
                def _():
                    gather_start(u + 1, other)

                gather_wait(u, b)
                transform(b)
                out_start(u, b)

        out_wait(units_per_w - 1, 1)

    return lookup_kernel


def kernel(x, table):
    b, t = x.shape
    xt = x.T.astype(jnp.int32)
    pairs = _make_pairs(table.T)
    out = _build_lookup(t, b)(xt, pairs)
    return out.transpose(2, 0, 1)


# speedup vs baseline: 1.6781x; 1.6781x over previous
import functools
import math

import jax
import jax.numpy as jnp
from jax import lax
from jax.experimental import pallas as pl
from jax.experimental.pallas import tpu as pltpu
from jax.experimental.pallas import tpu_sc as plsc

VOCAB = 1000000
DIM = 64
SCALE = math.sqrt(DIM)
LANES = 16
NUM_CORES = 2
NUM_SUBCORES = 16
NUM_WORKERS = NUM_CORES * NUM_SUBCORES
BBLK = 256
VBLK = 8192


def _pairs_body(tt_ref, p_ref):
    t8 = tt_ref[...].T * SCALE
    r3 = t8.reshape(VBLK // 2, 2, DIM)
    p_ref[...] = jnp.concatenate([r3[:, 0, :], r3[:, 1, :]], axis=1)


def _make_pairs(tt):
    grid = (pl.cdiv(VOCAB, VBLK),)
    return pl.pallas_call(
        _pairs_body,
        grid=grid,
        in_specs=[pl.BlockSpec((DIM, VBLK), lambda j: (0, j))],
        out_specs=pl.BlockSpec((VBLK // 2, 2 * DIM), lambda j: (j, 0)),
        out_shape=jax.ShapeDtypeStruct((VOCAB // 2, 2 * DIM), jnp.float32),
    )(tt)


@functools.lru_cache(maxsize=None)
def _build_lookup(seq: int, batch: int):
    n_bblk = batch // BBLK
    n_units = seq * n_bblk
    assert n_units % NUM_WORKERS == 0
    units_per_w = n_units // NUM_WORKERS
    assert units_per_w % 2 == 0

    mesh = plsc.VectorSubcoreMesh(
        core_axis_name="c", subcore_axis_name="s",
        num_cores=NUM_CORES, num_subcores=NUM_SUBCORES)

    @functools.partial(
        pl.kernel,
        out_type=jax.ShapeDtypeStruct((seq, DIM, batch), jnp.float32),
        mesh=mesh,
        scratch_types=[
            pltpu.VMEM((BBLK,), jnp.int32),
            pltpu.VMEM((BBLK,), jnp.int32),
            pltpu.VMEM((BBLK,), jnp.int32),
            pltpu.VMEM((BBLK,), jnp.int32),
            pltpu.VMEM((BBLK, 2 * DIM), jnp.float32),
            pltpu.VMEM((BBLK, 2 * DIM), jnp.float32),
            pltpu.VMEM((DIM, BBLK), jnp.float32),
            pltpu.VMEM((DIM, BBLK), jnp.float32),
            pltpu.SemaphoreType.DMA,
            pltpu.SemaphoreType.DMA,
            pltpu.SemaphoreType.DMA,
            pltpu.SemaphoreType.DMA,
        ],
        compiler_params=pltpu.CompilerParams(
            use_tc_tiling_on_sc=True, needs_layout_passes=False),
    )
    def lookup_kernel(xt_hbm, p_hbm, out_hbm, idx0, idx1, j0, j1,
                      rows0, rows1, tr0, tr1, gsem0, gsem1, osem0, osem1):
        idx = (idx0, idx1)
        jv = (j0, j1)
        rows = (rows0, rows1)
        tr = (tr0, tr1)
        gsem = (gsem0, gsem1)
        osem = (osem0, osem1)
        wid = lax.axis_index("s") * NUM_CORES + lax.axis_index("c")
        lane = lax.broadcasted_iota(jnp.int32, (LANES,), 0)

        def unit_tb(u):
            unit = wid * units_per_w + u
            return unit // n_bblk, (unit % n_bblk) * BBLK

        def gather_start(u, b):
            t, b0 = unit_tb(u)
            pltpu.sync_copy(xt_hbm.at[t, pl.ds(b0, BBLK)], idx[b])

            @pl.loop(0, BBLK // LANES)
            def _pairify(r):
                sl = pl.ds(r * LANES, LANES)
                jv[b][sl] = lax.shift_right_logical(idx[b][sl], 1)

            pltpu.async_copy(p_hbm.at[jv[b]], rows[b], gsem[b])

        def gather_wait(u, b):
            pltpu.make_async_copy(
                p_hbm.at[jv[b]], rows[b], gsem[b]).wait()

        def out_start(u, b):
            t, b0 = unit_tb(u)
            pltpu.async_copy(
                tr[b], out_hbm.at[t, :, pl.ds(b0, BBLK)], osem[b])

        def out_wait(u, b):
            t, b0 = unit_tb(u)
            pltpu.make_async_copy(
                tr[b], out_hbm.at[t, :, pl.ds(b0, BBLK)], osem[b]).wait()

        def transform(b):
            @plsc.parallel_loop(0, BBLK // LANES)
            def _rows(r):
                sl = pl.ds(r * LANES, LANES)
                rows16 = r * LANES + lane
                half16 = lax.shift_left(
                    lax.bitwise_and(idx[b][sl], jnp.int32(1)), 6)
                for d in range(DIM):
                    col16 = lax.bitwise_and(lane + d, jnp.int32(DIM - 1))
                    vals = plsc.load_gather(
                        rows[b], [rows16, half16 + col16])
                    plsc.store_scatter(tr[b], [col16, rows16], vals)

        gather_start(0, 0)

        @pl.loop(0, units_per_w, step=2)
        def _steady(outer):
            for b in range(2):
                u = outer + b
                other = 1 - b

                @pl.when(u > 0)
                def _():
                    out_wait(u - 1, other)

                @pl.when(u + 1 < units_per_w)
                def _():
                    gather_start(u + 1, other)

                gather_wait(u, b)
                transform(b)
                out_start(u, b)

        out_wait(units_per_w - 1, 1)

    return lookup_kernel


def kernel(x, table):
    b, t = x.shape
    xt = x.T.astype(jnp.int32)
    pairs = _make_pairs(table.T)
    out = _build_lookup(t, b)(xt, pairs)
    return out.transpose(2, 0, 1)


# speedup vs baseline: 2.0083x; 1.1967x over previous
import functools
import math

import jax
import jax.numpy as jnp
from jax import lax
from jax.experimental import pallas as pl
from jax.experimental.pallas import tpu as pltpu
from jax.experimental.pallas import tpu_sc as plsc

VOCAB = 1000000
DIM = 64
SCALE = math.sqrt(DIM)
LANES = 16
NUM_CORES = 2
NUM_SUBCORES = 16
NUM_WORKERS = NUM_CORES * NUM_SUBCORES
BBLK = 256
VBLK = 8192


N_VBLK = (VOCAB + VBLK - 1) // VBLK
P_ROWS = N_VBLK * (VBLK // 2)


def _pairs_body(tt_ref, p_ref):
    t8 = tt_ref[...].T * SCALE
    p_ref[...] = jnp.concatenate(
        [t8[: VBLK // 2], t8[VBLK // 2:]], axis=1)


def _make_pairs(tt):
    return pl.pallas_call(
        _pairs_body,
        grid=(N_VBLK,),
        in_specs=[pl.BlockSpec((DIM, VBLK), lambda j: (0, j))],
        out_specs=pl.BlockSpec((VBLK // 2, 2 * DIM), lambda j: (j, 0)),
        out_shape=jax.ShapeDtypeStruct((P_ROWS, 2 * DIM), jnp.float32),
    )(tt)


@functools.lru_cache(maxsize=None)
def _build_lookup(seq: int, batch: int):
    n_bblk = batch // BBLK
    n_units = seq * n_bblk
    assert n_units % NUM_WORKERS == 0
    units_per_w = n_units // NUM_WORKERS
    assert units_per_w % 2 == 0

    mesh = plsc.VectorSubcoreMesh(
        core_axis_name="c", subcore_axis_name="s",
        num_cores=NUM_CORES, num_subcores=NUM_SUBCORES)

    @functools.partial(
        pl.kernel,
        out_type=jax.ShapeDtypeStruct((seq, DIM, batch), jnp.float32),
        mesh=mesh,
        scratch_types=[
            pltpu.VMEM((BBLK,), jnp.int32),
            pltpu.VMEM((BBLK,), jnp.int32),
            pltpu.VMEM((BBLK,), jnp.int32),
            pltpu.VMEM((BBLK,), jnp.int32),
            pltpu.VMEM((BBLK, 2 * DIM), jnp.float32),
            pltpu.VMEM((BBLK, 2 * DIM), jnp.float32),
            pltpu.VMEM((DIM, BBLK), jnp.float32),
            pltpu.VMEM((DIM, BBLK), jnp.float32),
            pltpu.SemaphoreType.DMA,
            pltpu.SemaphoreType.DMA,
            pltpu.SemaphoreType.DMA,
            pltpu.SemaphoreType.DMA,
        ],
        compiler_params=pltpu.CompilerParams(
            use_tc_tiling_on_sc=True, needs_layout_passes=False),
    )
    def lookup_kernel(xt_hbm, p_hbm, out_hbm, idx0, idx1, j0, j1,
                      rows0, rows1, tr0, tr1, gsem0, gsem1, osem0, osem1):
        idx = (idx0, idx1)
        jv = (j0, j1)
        rows = (rows0, rows1)
        tr = (tr0, tr1)
        gsem = (gsem0, gsem1)
        osem = (osem0, osem1)
        wid = lax.axis_index("s") * NUM_CORES + lax.axis_index("c")
        lane = lax.broadcasted_iota(jnp.int32, (LANES,), 0)

        def unit_tb(u):
            unit = wid * units_per_w + u
            return unit // n_bblk, (unit % n_bblk) * BBLK

        def gather_start(u, b):
            t, b0 = unit_tb(u)
            pltpu.sync_copy(xt_hbm.at[t, pl.ds(b0, BBLK)], idx[b])

            @pl.loop(0, BBLK // LANES)
            def _pairify(r):
                sl = pl.ds(r * LANES, LANES)
                v16 = idx[b][sl]
                jv[b][sl] = lax.bitwise_or(
                    lax.shift_left(lax.shift_right_logical(v16, 13), 12),
                    lax.bitwise_and(v16, jnp.int32(VBLK // 2 - 1)))

            pltpu.async_copy(p_hbm.at[jv[b]], rows[b], gsem[b])

        def gather_wait(u, b):
            pltpu.make_async_copy(
                p_hbm.at[jv[b]], rows[b], gsem[b]).wait()

        def out_start(u, b):
            t, b0 = unit_tb(u)
            pltpu.async_copy(
                tr[b], out_hbm.at[t, :, pl.ds(b0, BBLK)], osem[b])

        def out_wait(u, b):
            t, b0 = unit_tb(u)
            pltpu.make_async_copy(
                tr[b], out_hbm.at[t, :, pl.ds(b0, BBLK)], osem[b]).wait()

        def transform(b):
            @plsc.parallel_loop(0, BBLK // LANES)
            def _rows(r):
                sl = pl.ds(r * LANES, LANES)
                rows16 = r * LANES + lane
                half16 = lax.shift_left(
                    lax.bitwise_and(
                        lax.shift_right_logical(idx[b][sl], 12),
                        jnp.int32(1)), 6)
                for d in range(DIM):
                    col16 = lax.bitwise_and(lane + d, jnp.int32(DIM - 1))
                    vals = plsc.load_gather(
                        rows[b], [rows16, half16 + col16])
                    plsc.store_scatter(tr[b], [col16, rows16], vals)

        gather_start(0, 0)

        @pl.loop(0, units_per_w, step=2)
        def _steady(outer):
            for b in range(2):
                u = outer + b
                other = 1 - b

                @pl.when(u > 0)
                def _():
                    out_wait(u - 1, other)

                @pl.when(u + 1 < units_per_w)
                def _():
                    gather_start(u + 1, other)

                gather_wait(u, b)
                transform(b)
                out_start(u, b)

        out_wait(units_per_w - 1, 1)

    return lookup_kernel


def kernel(x, table):
    b, t = x.shape
    xt = x.T.astype(jnp.int32)
    pairs = _make_pairs(table.T)
    out = _build_lookup(t, b)(xt, pairs)
    return out.transpose(2, 0, 1)


# speedup vs baseline: 2.1016x; 1.0465x over previous
import functools
import math

import jax
import jax.numpy as jnp
from jax import lax
from jax.experimental import pallas as pl
from jax.experimental.pallas import tpu as pltpu
from jax.experimental.pallas import tpu_sc as plsc

VOCAB = 1000000
DIM = 64
SCALE = math.sqrt(DIM)
LANES = 16
NUM_CORES = 2
NUM_SUBCORES = 16
NUM_WORKERS = NUM_CORES * NUM_SUBCORES
BBLK = 256
VBLK = 16384


N_VBLK = (VOCAB + VBLK - 1) // VBLK
P_ROWS = N_VBLK * (VBLK // 2)


def _pairs_body(tt_ref, p_ref):
    t8 = tt_ref[...].T * SCALE
    p_ref[...] = jnp.concatenate(
        [t8[: VBLK // 2], t8[VBLK // 2:]], axis=1)


def _make_pairs(tt):
    return pl.pallas_call(
        _pairs_body,
        grid=(N_VBLK,),
        in_specs=[pl.BlockSpec((DIM, VBLK), lambda j: (0, j))],
        out_specs=pl.BlockSpec((VBLK // 2, 2 * DIM), lambda j: (j, 0)),
        out_shape=jax.ShapeDtypeStruct((P_ROWS, 2 * DIM), jnp.float32),
    )(tt)


@functools.lru_cache(maxsize=None)
def _build_lookup(seq: int, batch: int):
    n_bblk = batch // BBLK
    n_units = seq * n_bblk
    assert n_units % NUM_WORKERS == 0
    units_per_w = n_units // NUM_WORKERS
    assert units_per_w % 2 == 0

    mesh = plsc.VectorSubcoreMesh(
        core_axis_name="c", subcore_axis_name="s",
        num_cores=NUM_CORES, num_subcores=NUM_SUBCORES)

    @functools.partial(
        pl.kernel,
        out_type=jax.ShapeDtypeStruct((seq, DIM, batch), jnp.float32),
        mesh=mesh,
        scratch_types=[
            pltpu.VMEM((BBLK,), jnp.int32),
            pltpu.VMEM((BBLK,), jnp.int32),
            pltpu.VMEM((BBLK,), jnp.int32),
            pltpu.VMEM((BBLK,), jnp.int32),
            pltpu.VMEM((BBLK, 2 * DIM), jnp.float32),
            pltpu.VMEM((BBLK, 2 * DIM), jnp.float32),
            pltpu.VMEM((DIM, BBLK), jnp.float32),
            pltpu.VMEM((DIM, BBLK), jnp.float32),
            pltpu.SemaphoreType.DMA,
            pltpu.SemaphoreType.DMA,
            pltpu.SemaphoreType.DMA,
            pltpu.SemaphoreType.DMA,
        ],
        compiler_params=pltpu.CompilerParams(
            use_tc_tiling_on_sc=True, needs_layout_passes=False),
    )
    def lookup_kernel(xt_hbm, p_hbm, out_hbm, idx0, idx1, j0, j1,
                      rows0, rows1, tr0, tr1, gsem0, gsem1, osem0, osem1):
        idx = (idx0, idx1)
        jv = (j0, j1)
        rows = (rows0, rows1)
        tr = (tr0, tr1)
        gsem = (gsem0, gsem1)
        osem = (osem0, osem1)
        wid = lax.axis_index("s") * NUM_CORES + lax.axis_index("c")
        lane = lax.broadcasted_iota(jnp.int32, (LANES,), 0)

        def unit_tb(u):
            unit = wid * units_per_w + u
            return unit // n_bblk, (unit % n_bblk) * BBLK

        def gather_start(u, b):
            t, b0 = unit_tb(u)
            pltpu.sync_copy(xt_hbm.at[t, pl.ds(b0, BBLK)], idx[b])

            @pl.loop(0, BBLK // LANES)
            def _pairify(r):
                sl = pl.ds(r * LANES, LANES)
                v16 = idx[b][sl]
                jv[b][sl] = lax.bitwise_or(
                    lax.shift_left(lax.shift_right_logical(v16, 14), 13),
                    lax.bitwise_and(v16, jnp.int32(VBLK // 2 - 1)))

            pltpu.async_copy(p_hbm.at[jv[b]], rows[b], gsem[b])

        def gather_wait(u, b):
            pltpu.make_async_copy(
                p_hbm.at[jv[b]], rows[b], gsem[b]).wait()

        def out_start(u, b):
            t, b0 = unit_tb(u)
            pltpu.async_copy(
                tr[b], out_hbm.at[t, :, pl.ds(b0, BBLK)], osem[b])

        def out_wait(u, b):
            t, b0 = unit_tb(u)
            pltpu.make_async_copy(
                tr[b], out_hbm.at[t, :, pl.ds(b0, BBLK)], osem[b]).wait()

        def transform(b):
            @plsc.parallel_loop(0, BBLK // LANES)
            def _rows(r):
                sl = pl.ds(r * LANES, LANES)
                rows16 = r * LANES + lane
                half16 = lax.shift_left(
                    lax.bitwise_and(
                        lax.shift_right_logical(idx[b][sl], 13),
                        jnp.int32(1)), 6)
                for d in range(DIM):
                    col16 = lax.bitwise_and(lane + d, jnp.int32(DIM - 1))
                    vals = plsc.load_gather(
                        rows[b], [rows16, half16 + col16])
                    plsc.store_scatter(tr[b], [col16, rows16], vals)

        gather_start(0, 0)

        @pl.loop(0, units_per_w, step=2)
        def _steady(outer):
            for b in range(2):
                u = outer + b
                other = 1 - b

                @pl.when(u > 0)
                def _():
                    out_wait(u - 1, other)

                @pl.when(u + 1 < units_per_w)
                def _():
                    gather_start(u + 1, other)

                gather_wait(u, b)
                transform(b)
                out_start(u, b)

        out_wait(units_per_w - 1, 1)

    return lookup_kernel


def kernel(x, table):
    b, t = x.shape
    xt = x.T.astype(jnp.int32)
    pairs = _make_pairs(table.T)
    out = _build_lookup(t, b)(xt, pairs)
    return out.transpose(2, 0, 1)


# speedup vs baseline: 2.2130x; 1.0530x over previous
import functools
import math

import jax
import jax.numpy as jnp
from jax import lax
from jax.experimental import pallas as pl
from jax.experimental.pallas import tpu as pltpu
from jax.experimental.pallas import tpu_sc as plsc

VOCAB = 1000000
DIM = 64
SCALE = math.sqrt(DIM)
LANES = 16
NUM_CORES = 2
NUM_SUBCORES = 16
NUM_WORKERS = NUM_CORES * NUM_SUBCORES
BBLK = 256
VBLK = 16384


N_VBLK = (VOCAB + VBLK - 1) // VBLK
P_ROWS = N_VBLK * (VBLK // 2)


def _pairs_body(tt_ref, p_ref):
    t8 = tt_ref[...].T * SCALE
    p_ref[...] = jnp.concatenate(
        [t8[: VBLK // 2], t8[VBLK // 2:]], axis=1)


def _make_pairs(tt):
    return pl.pallas_call(
        _pairs_body,
        grid=(N_VBLK,),
        in_specs=[pl.BlockSpec((DIM, VBLK), lambda j: (0, j))],
        out_specs=pl.BlockSpec((VBLK // 2, 2 * DIM), lambda j: (j, 0)),
        out_shape=jax.ShapeDtypeStruct((P_ROWS, 2 * DIM), jnp.float32),
    )(tt)


@functools.lru_cache(maxsize=None)
def _build_lookup(seq: int, batch: int):
    n_bblk = batch // BBLK
    n_units = seq * n_bblk
    assert n_units % NUM_WORKERS == 0
    units_per_w = n_units // NUM_WORKERS
    assert units_per_w % 2 == 0

    mesh = plsc.VectorSubcoreMesh(
        core_axis_name="c", subcore_axis_name="s",
        num_cores=NUM_CORES, num_subcores=NUM_SUBCORES)

    @functools.partial(
        pl.kernel,
        out_type=jax.ShapeDtypeStruct((seq, DIM, batch), jnp.float32),
        mesh=mesh,
        scratch_types=[
            pltpu.VMEM((BBLK,), jnp.int32),
            pltpu.VMEM((BBLK,), jnp.int32),
            pltpu.VMEM((BBLK,), jnp.int32),
            pltpu.VMEM((BBLK,), jnp.int32),
            pltpu.VMEM((BBLK, 2 * DIM), jnp.float32),
            pltpu.VMEM((BBLK, 2 * DIM), jnp.float32),
            pltpu.VMEM((DIM, BBLK), jnp.float32),
            pltpu.VMEM((DIM, BBLK), jnp.float32),
            pltpu.SemaphoreType.DMA,
            pltpu.SemaphoreType.DMA,
            pltpu.SemaphoreType.DMA,
            pltpu.SemaphoreType.DMA,
        ],
        compiler_params=pltpu.CompilerParams(
            use_tc_tiling_on_sc=True, needs_layout_passes=False),
    )
    def lookup_kernel(xt_hbm, p_hbm, out_hbm, idx0, idx1, j0, j1,
                      rows0, rows1, tr0, tr1, gsem0, gsem1, osem0, osem1):
        idx = (idx0, idx1)
        jv = (j0, j1)
        rows = (rows0, rows1)
        tr = (tr0, tr1)
        gsem = (gsem0, gsem1)
        osem = (osem0, osem1)
        wid = lax.axis_index("s") * NUM_CORES + lax.axis_index("c")
        lane = lax.broadcasted_iota(jnp.int32, (LANES,), 0)

        def unit_tb(u):
            unit = wid * units_per_w + u
            return unit // n_bblk, (unit % n_bblk) * BBLK

        def gather_start(u, b):
            t, b0 = unit_tb(u)
            pltpu.sync_copy(xt_hbm.at[t, pl.ds(b0, BBLK)], idx[b])

            @pl.loop(0, BBLK // LANES)
            def _pairify(r):
                sl = pl.ds(r * LANES, LANES)
                v16 = idx[b][sl]
                jv[b][sl] = lax.bitwise_or(
                    lax.shift_left(lax.shift_right_logical(v16, 14), 13),
                    lax.bitwise_and(v16, jnp.int32(VBLK // 2 - 1)))

            pltpu.async_copy(p_hbm.at[jv[b]], rows[b], gsem[b])

        def gather_wait(u, b):
            pltpu.make_async_copy(
                p_hbm.at[jv[b]], rows[b], gsem[b]).wait()

        def out_start(u, b):
            t, b0 = unit_tb(u)
            pltpu.async_copy(
                tr[b], out_hbm.at[t, :, pl.ds(b0, BBLK)], osem[b])

        def out_wait(u, b):
            t, b0 = unit_tb(u)
            pltpu.make_async_copy(
                tr[b], out_hbm.at[t, :, pl.ds(b0, BBLK)], osem[b]).wait()

        def transform(b):
            @plsc.parallel_loop(0, BBLK // LANES, unroll=2)
            def _rows(r):
                sl = pl.ds(r * LANES, LANES)
                rows16 = r * LANES + lane
                half16 = lax.shift_left(
                    lax.bitwise_and(
                        lax.shift_right_logical(idx[b][sl], 13),
                        jnp.int32(1)), 6)
                for d in range(DIM):
                    col16 = lax.bitwise_and(lane + d, jnp.int32(DIM - 1))
                    vals = plsc.load_gather(
                        rows[b], [rows16, half16 + col16])
                    plsc.store_scatter(tr[b], [col16, rows16], vals)

        gather_start(0, 0)

        @pl.loop(0, units_per_w, step=2)
        def _steady(outer):
            for b in range(2):
                u = outer + b
                other = 1 - b

                @pl.when(u > 0)
                def _():
                    out_wait(u - 1, other)

                @pl.when(u + 1 < units_per_w)
                def _():
                    gather_start(u + 1, other)

                gather_wait(u, b)
                transform(b)
                out_start(u, b)

        out_wait(units_per_w - 1, 1)

    return lookup_kernel


def kernel(x, table):
    b, t = x.shape
    xt = x.T.astype(jnp.int32)
    pairs = _make_pairs(table.T)
    out = _build_lookup(t, b)(xt, pairs)
    return out.transpose(2, 0, 1)


# speedup vs baseline: 2.8065x; 1.2682x over previous
import functools
import math

import jax
import jax.numpy as jnp
from jax import lax
from jax.experimental import pallas as pl
from jax.experimental.pallas import tpu as pltpu
from jax.experimental.pallas import tpu_sc as plsc

VOCAB = 1000000
DIM = 64
SCALE = math.sqrt(DIM)
LANES = 16
NUM_CORES = 2
NUM_SUBCORES = 16
NUM_WORKERS = NUM_CORES * NUM_SUBCORES
BBLK = 256
VBLK = 16384


N_VBLK = (VOCAB + VBLK - 1) // VBLK
P_ROWS = N_VBLK * (VBLK // 2)


def _pairs_body(tt_ref, p_ref):
    t8 = tt_ref[...].T * SCALE
    p_ref[...] = jnp.concatenate(
        [t8[: VBLK // 2], t8[VBLK // 2:]], axis=1)


def _make_pairs(tt):
    return pl.pallas_call(
        _pairs_body,
        grid=(N_VBLK,),
        in_specs=[pl.BlockSpec((DIM, VBLK), lambda j: (0, j))],
        out_specs=pl.BlockSpec((VBLK // 2, 2 * DIM), lambda j: (j, 0)),
        out_shape=jax.ShapeDtypeStruct((P_ROWS, 2 * DIM), jnp.float32),
    )(tt)


@functools.lru_cache(maxsize=None)
def _build_lookup(seq: int, batch: int):
    n_bblk = batch // BBLK
    n_units = seq * n_bblk
    assert n_units % NUM_WORKERS == 0
    units_per_w = n_units // NUM_WORKERS
    assert units_per_w % 2 == 0

    mesh = plsc.VectorSubcoreMesh(
        core_axis_name="c", subcore_axis_name="s",
        num_cores=NUM_CORES, num_subcores=NUM_SUBCORES)

    @functools.partial(
        pl.kernel,
        out_type=jax.ShapeDtypeStruct((seq, DIM, batch), jnp.float32),
        mesh=mesh,
        scratch_types=[
            pltpu.VMEM((BBLK,), jnp.int32),
            pltpu.VMEM((BBLK,), jnp.int32),
            pltpu.VMEM((BBLK,), jnp.int32),
            pltpu.VMEM((BBLK,), jnp.int32),
            pltpu.VMEM((BBLK, 2 * DIM), jnp.float32),
            pltpu.VMEM((BBLK, 2 * DIM), jnp.float32),
            pltpu.VMEM((DIM, BBLK), jnp.float32),
            pltpu.VMEM((DIM, BBLK), jnp.float32),
            pltpu.SemaphoreType.DMA,
            pltpu.SemaphoreType.DMA,
            pltpu.SemaphoreType.DMA,
            pltpu.SemaphoreType.DMA,
        ],
        compiler_params=pltpu.CompilerParams(
            use_tc_tiling_on_sc=True, needs_layout_passes=False),
    )
    def lookup_kernel(xt_hbm, p_hbm, out_hbm, idx0, idx1, j0, j1,
                      rows0, rows1, tr0, tr1, gsem0, gsem1, osem0, osem1):
        idx = (idx0, idx1)
        jv = (j0, j1)
        rows = (rows0, rows1)
        tr = (tr0, tr1)
        gsem = (gsem0, gsem1)
        osem = (osem0, osem1)
        wid = lax.axis_index("s") * NUM_CORES + lax.axis_index("c")
        lane = lax.broadcasted_iota(jnp.int32, (LANES,), 0)

        def unit_tb(u):
            unit = wid * units_per_w + u
            return unit // n_bblk, (unit % n_bblk) * BBLK

        def gather_start(u, b):
            t, b0 = unit_tb(u)
            pltpu.sync_copy(xt_hbm.at[t, pl.ds(b0, BBLK)], idx[b])

            @pl.loop(0, BBLK // LANES)
            def _pairify(r):
                sl = pl.ds(r * LANES, LANES)
                v16 = idx[b][sl]
                jv[b][sl] = lax.bitwise_or(
                    lax.shift_left(lax.shift_right_logical(v16, 14), 13),
                    lax.bitwise_and(v16, jnp.int32(VBLK // 2 - 1)))

            pltpu.async_copy(p_hbm.at[jv[b]], rows[b], gsem[b])

        def gather_wait(u, b):
            pltpu.make_async_copy(
                p_hbm.at[jv[b]], rows[b], gsem[b]).wait()

        def out_start(u, b):
            t, b0 = unit_tb(u)
            pltpu.async_copy(
                tr[b], out_hbm.at[t, :, pl.ds(b0, BBLK)], osem[b])

        def out_wait(u, b):
            t, b0 = unit_tb(u)
            pltpu.make_async_copy(
                tr[b], out_hbm.at[t, :, pl.ds(b0, BBLK)], osem[b]).wait()

        def transform(b):
            @plsc.parallel_loop(0, BBLK // LANES, unroll=4)
            def _rows(r):
                sl = pl.ds(r * LANES, LANES)
                rows16 = r * LANES + lane
                half16 = lax.shift_left(
                    lax.bitwise_and(
                        lax.shift_right_logical(idx[b][sl], 13),
                        jnp.int32(1)), 6)
                for d in range(DIM):
                    col16 = lax.bitwise_and(lane + d, jnp.int32(DIM - 1))
                    vals = plsc.load_gather(
                        rows[b], [rows16, half16 + col16])
                    plsc.store_scatter(tr[b], [col16, rows16], vals)

        gather_start(0, 0)

        @pl.loop(0, units_per_w, step=2)
        def _steady(outer):
            for b in range(2):
                u = outer + b
                other = 1 - b

                @pl.when(u > 0)
                def _():
                    out_wait(u - 1, other)

                @pl.when(u + 1 < units_per_w)
                def _():
                    gather_start(u + 1, other)

                gather_wait(u, b)
                transform(b)
                out_start(u, b)

        out_wait(units_per_w - 1, 1)

    return lookup_kernel


def kernel(x, table):
    b, t = x.shape
    xt = x.T.astype(jnp.int32)
    pairs = _make_pairs(table.T)
    out = _build_lookup(t, b)(xt, pairs)
    return out.transpose(2, 0, 1)
